# baseline (speedup 1.0000x reference)
import functools

import jax
import jax.numpy as jnp
from jax import lax
from jax.experimental import pallas as pl
from jax.experimental.pallas import tpu as pltpu
from jax.experimental.pallas import tpu_sc as plsc

N_NODES = 10000
N_EDGES = 320000
IN_FEATS = 128
HIDDEN = 128
N_CLASSES = 64
K = 10
ALPHA = 0.1

NC = 2
NS = 16
LANES = 16
N_PAD = 10240
ROWS = N_PAD // NS
EB = 128
NB = 160
EPT = NB * EB
E_PAD = NS * EPT
COLS = N_CLASSES // NC
ZROWS = 64

_mesh = plsc.VectorSubcoreMesh(core_axis_name="c", subcore_axis_name="s")
_sc_params = pltpu.CompilerParams(use_tc_tiling_on_sc=False)


def _zero_vmem(ref, rows, cols):
    for r in range(rows):
        for h in range(cols // LANES):
            ref[r, pl.ds(h * LANES, LANES)] = jnp.zeros((LANES,), jnp.float32)


@functools.partial(
    pl.kernel,
    out_type=jax.ShapeDtypeStruct((NC, N_PAD), jnp.float32),
    mesh=_mesh,
    compiler_params=_sc_params,
    scratch_types=[
        pltpu.VMEM_SHARED((N_PAD,), jnp.float32),
        pltpu.VMEM((NB, EB), jnp.int32),
        pltpu.VMEM((EB,), jnp.float32),
        pltpu.VMEM((ROWS,), jnp.float32),
    ],
)
def _degrees_kernel(idx_hbm, out_hbm, deg_sp, idx_v, ones_v, stage_v):
    c = lax.axis_index("c")
    s = lax.axis_index("s")
    r0 = s * ROWS
    for h in range(EB // LANES):
        ones_v[pl.ds(h * LANES, LANES)] = jnp.ones((LANES,), jnp.float32)
    for h in range(ROWS // LANES):
        stage_v[pl.ds(h * LANES, LANES)] = jnp.zeros((LANES,), jnp.float32)
    pltpu.sync_copy(stage_v, deg_sp.at[pl.ds(r0, ROWS)])
    pltpu.sync_copy(idx_hbm.at[c, s], idx_v)
    plsc.subcore_barrier()

    def body(j, carry):
        pltpu.sync_copy(ones_v, deg_sp.at[idx_v.at[j]], add=True)
        return carry

    lax.fori_loop(0, NB, body, 0)
    plsc.subcore_barrier()
    pltpu.sync_copy(deg_sp.at[pl.ds(r0, ROWS)], stage_v)
    pltpu.sync_copy(stage_v, out_hbm.at[c, pl.ds(r0, ROWS)])


def _mlp_body(x_ref, w0_ref, b0_ref, w1_ref, b1_ref, w2_ref, b2_ref, o_ref):
    x = x_ref[...]
    h = jnp.dot(x, w0_ref[...], preferred_element_type=jnp.float32) + b0_ref[...]
    h = jnp.maximum(h, 0.0)
    h = jnp.dot(h, w1_ref[...], preferred_element_type=jnp.float32) + b1_ref[...]
    h = jnp.maximum(h, 0.0)
    o_ref[...] = jnp.dot(h, w2_ref[...], preferred_element_type=jnp.float32) + b2_ref[...]


_MLP_BLK = 1024


def _mlp(x_pad, W0, b0, W1, b1, W2, b2):
    grid = (N_PAD // _MLP_BLK,)
    return pl.pallas_call(
        _mlp_body,
        grid=grid,
        in_specs=[
            pl.BlockSpec((_MLP_BLK, IN_FEATS), lambda i: (i, 0)),
            pl.BlockSpec((IN_FEATS, HIDDEN), lambda i: (0, 0)),
            pl.BlockSpec((HIDDEN,), lambda i: (0,)),
            pl.BlockSpec((HIDDEN, HIDDEN), lambda i: (0, 0)),
            pl.BlockSpec((HIDDEN,), lambda i: (0,)),
            pl.BlockSpec((HIDDEN, N_CLASSES), lambda i: (0, 0)),
            pl.BlockSpec((N_CLASSES,), lambda i: (0,)),
        ],
        out_specs=pl.BlockSpec((_MLP_BLK, N_CLASSES), lambda i: (i, 0)),
        out_shape=jax.ShapeDtypeStruct((N_PAD, N_CLASSES), jnp.float32),
    )(x_pad, W0, b0, W1, b1, W2, b2)


def _axpy_chunk(p_ref, a_ref, w_ref):
    def body(r, carry):
        for h in range(COLS // LANES):
            sl = pl.ds(h * LANES, LANES)
            w_ref[r, sl] = p_ref[r, sl] * w_ref[r, sl] + a_ref[r, sl]
        return carry

    lax.fori_loop(0, ZROWS, body, 0)


@functools.partial(
    pl.kernel,
    out_type=[jax.ShapeDtypeStruct((NC, N_PAD, COLS), jnp.float32),
              jax.ShapeDtypeStruct((NC, N_PAD, COLS), jnp.float32)],
    mesh=_mesh,
    compiler_params=_sc_params,
    scratch_types=[
        pltpu.VMEM_SHARED((N_PAD, COLS), jnp.float32),
        pltpu.VMEM((NB, EB), jnp.int32),
        pltpu.VMEM((NB, EB), jnp.int32),
        [pltpu.VMEM((EB, COLS), jnp.float32)] * 8,
        pltpu.VMEM((ZROWS, COLS), jnp.float32),
        pltpu.VMEM((ZROWS, COLS), jnp.float32),
        pltpu.VMEM((ZROWS, COLS), jnp.float32),
        pltpu.VMEM((ZROWS, COLS), jnp.float32),
        pltpu.SemaphoreType.DMA((8,)),
        pltpu.SemaphoreType.DMA((8,)),
    ],
)
def _prop_kernel(src_hbm, dst_hbm, g0_hbm, a0_hbm, p2_hbm, q2_hbm, h0a_hbm,
                 out_hbm, gw_hbm, agg_sp, src_v, dst_v, gat, w_v, p_v, a_v,
                 z_v, gsem, ssem):
    c = lax.axis_index("c")
    s = lax.axis_index("s")
    r0 = s * ROWS
    nz = ROWS // ZROWS

    pltpu.sync_copy(src_hbm.at[s], src_v)
    pltpu.sync_copy(dst_hbm.at[s], dst_v)
    _zero_vmem(z_v, ZROWS, COLS)

    def zero_body(z, carry):
        pltpu.sync_copy(z_v, agg_sp.at[pl.ds(r0 + z * ZROWS, ZROWS)])
        return carry

    lax.fori_loop(0, nz, zero_body, 0)
    plsc.subcore_barrier()

    def edge_phase(gsrc):
        def g_issue(j, b):
            pltpu.async_copy(gsrc.at[c, :, :].at[src_v.at[j]], gat[b],
                             gsem.at[b])

        def g_wait(b):
            pltpu.make_async_copy(gsrc.at[c, :, :].at[src_v.at[0]], gat[b],
                                  gsem.at[b]).wait()

        def s_issue(j, b):
            pltpu.async_copy(gat[b], agg_sp.at[dst_v.at[j]], ssem.at[b],
                             add=True)

        def s_wait(b):
            pltpu.make_async_copy(gat[b], agg_sp.at[dst_v.at[0]],
                                  ssem.at[b]).wait()

        for b in range(4):
            g_issue(b, b)
        for b in range(8):
            g_wait(b)
            s_issue(b, b)
            if b >= 4:
                s_wait(b - 4)
            g_issue(b + 4, (b + 4) % 8)

        def group(g, carry):
            for b in range(8):
                v = 8 * g + b
                g_wait(b)
                s_issue(v, b)
                s_wait((b + 4) % 8)
                g_issue(jnp.minimum(v + 4, NB - 1), (b + 4) % 8)
            return carry

        lax.fori_loop(1, NB // 8, group, 0)
        for b in range(4):
            g_wait(b)
        for b in range(4, 8):
            s_wait(b)

    for k in range(K):
        edge_phase(g0_hbm if k == 0 else gw_hbm)
        plsc.subcore_barrier()
        last = k == K - 1
        pm_hbm = q2_hbm if last else p2_hbm
        am_hbm = h0a_hbm if last else a0_hbm

        def upd_body(z, carry, last=last, pm_hbm=pm_hbm, am_hbm=am_hbm):
            zsl = pl.ds(r0 + z * ZROWS, ZROWS)
            pltpu.sync_copy(agg_sp.at[zsl], w_v)
            if not last:
                pltpu.sync_copy(z_v, agg_sp.at[zsl])
            pltpu.sync_copy(pm_hbm.at[c, zsl], p_v)
            pltpu.sync_copy(am_hbm.at[c, zsl], a_v)
            _axpy_chunk(p_v, a_v, w_v)
            if last:
                pltpu.sync_copy(w_v, out_hbm.at[c, zsl])
            else:
                pltpu.sync_copy(w_v, gw_hbm.at[c, zsl])
            return carry

        lax.fori_loop(0, nz, upd_body, 0)
        if not last:
            plsc.subcore_barrier()


def kernel(features, edge_index, W0, b0, W1, b1, W2, b2):
    f32 = jnp.float32
    src = edge_index[0]
    dst = edge_index[1]
    pad_e = jnp.full((E_PAD - N_EDGES,), N_NODES, jnp.int32)
    src_t = jnp.concatenate([src, pad_e]).reshape(NS, NB, EB)
    dst_t = jnp.concatenate([dst, pad_e]).reshape(NS, NB, EB)

    degs = _degrees_kernel(jnp.stack([src_t, dst_t]))
    out_deg = degs[0]
    in_deg = degs[1]
    out_norm = jnp.power(jnp.clip(out_deg, 1.0, None), -0.5)
    in_norm = jnp.power(jnp.clip(in_deg, 1.0, None), -0.5)

    x_pad = jnp.zeros((N_PAD, IN_FEATS), f32).at[:N_NODES].set(features)
    h0 = _mlp(x_pad, W0, b0, W1, b1, W2, b2)
    mask = (jnp.arange(N_PAD) < N_NODES).astype(f32)
    h0 = h0 * mask[:, None]

    g0 = h0 * out_norm[:, None]
    a0 = ALPHA * g0
    p2 = jnp.broadcast_to(((1.0 - ALPHA) * out_norm * in_norm)[:, None],
                          (N_PAD, N_CLASSES))
    q2 = jnp.broadcast_to(((1.0 - ALPHA) * in_norm)[:, None],
                          (N_PAD, N_CLASSES))
    h0a = ALPHA * h0

    def split_cols(x):
        return jnp.stack([x[:, :COLS], x[:, COLS:]])

    out, _ = _prop_kernel(src_t, dst_t, split_cols(g0), split_cols(a0),
                          split_cols(p2), split_cols(q2), split_cols(h0a))
    return jnp.concatenate([out[0, :N_NODES], out[1, :N_NODES]], axis=1)


# speedup vs baseline: 1.6778x; 1.6778x over previous
import functools

import jax
import jax.numpy as jnp
from jax import lax
from jax.experimental import pallas as pl
from jax.experimental.pallas import tpu as pltpu
from jax.experimental.pallas import tpu_sc as plsc

N_NODES = 10000
N_EDGES = 320000
IN_FEATS = 128
HIDDEN = 128
N_CLASSES = 64
K = 10
ALPHA = 0.1

NC = 2
NS = 16
LANES = 16
N_PAD = 10240
ROWS = N_PAD // NS
EB = 128
NB = 160
EPT = NB * EB
E_PAD = NS * EPT
COLS = N_CLASSES // NC
ZROWS = 64

_mesh = plsc.VectorSubcoreMesh(core_axis_name="c", subcore_axis_name="s")
_sc_params = pltpu.CompilerParams(use_tc_tiling_on_sc=False)


def _zero_vmem(ref, rows, cols):
    for r in range(rows):
        for h in range(cols // LANES):
            ref[r, pl.ds(h * LANES, LANES)] = jnp.zeros((LANES,), jnp.float32)


@functools.partial(
    pl.kernel,
    out_type=jax.ShapeDtypeStruct((NC, N_PAD), jnp.float32),
    mesh=_mesh,
    compiler_params=_sc_params,
    scratch_types=[
        pltpu.VMEM_SHARED((N_PAD,), jnp.float32),
        pltpu.VMEM((NB, EB), jnp.int32),
        pltpu.VMEM((EB,), jnp.float32),
        pltpu.VMEM((ROWS,), jnp.float32),
    ],
)
def _degrees_kernel(idx_hbm, out_hbm, deg_sp, idx_v, ones_v, stage_v):
    c = lax.axis_index("c")
    s = lax.axis_index("s")
    r0 = s * ROWS
    for h in range(EB // LANES):
        ones_v[pl.ds(h * LANES, LANES)] = jnp.ones((LANES,), jnp.float32)
    for h in range(ROWS // LANES):
        stage_v[pl.ds(h * LANES, LANES)] = jnp.zeros((LANES,), jnp.float32)
    pltpu.sync_copy(stage_v, deg_sp.at[pl.ds(r0, ROWS)])
    pltpu.sync_copy(idx_hbm.at[c, s], idx_v)
    plsc.subcore_barrier()

    def body(j, carry):
        pltpu.sync_copy(ones_v, deg_sp.at[idx_v.at[j]], add=True)
        return carry

    lax.fori_loop(0, NB, body, 0)
    plsc.subcore_barrier()
    pltpu.sync_copy(deg_sp.at[pl.ds(r0, ROWS)], stage_v)
    pltpu.sync_copy(stage_v, out_hbm.at[c, pl.ds(r0, ROWS)])


def _mlp_body(x_ref, w0_ref, b0_ref, w1_ref, b1_ref, w2_ref, b2_ref, o_ref):
    x = x_ref[...]
    h = jnp.dot(x, w0_ref[...], preferred_element_type=jnp.float32) + b0_ref[...]
    h = jnp.maximum(h, 0.0)
    h = jnp.dot(h, w1_ref[...], preferred_element_type=jnp.float32) + b1_ref[...]
    h = jnp.maximum(h, 0.0)
    o_ref[...] = jnp.dot(h, w2_ref[...], preferred_element_type=jnp.float32) + b2_ref[...]


_MLP_BLK = 1024


def _mlp(x_pad, W0, b0, W1, b1, W2, b2):
    grid = (N_PAD // _MLP_BLK,)
    return pl.pallas_call(
        _mlp_body,
        grid=grid,
        in_specs=[
            pl.BlockSpec((_MLP_BLK, IN_FEATS), lambda i: (i, 0)),
            pl.BlockSpec((IN_FEATS, HIDDEN), lambda i: (0, 0)),
            pl.BlockSpec((HIDDEN,), lambda i: (0,)),
            pl.BlockSpec((HIDDEN, HIDDEN), lambda i: (0, 0)),
            pl.BlockSpec((HIDDEN,), lambda i: (0,)),
            pl.BlockSpec((HIDDEN, N_CLASSES), lambda i: (0, 0)),
            pl.BlockSpec((N_CLASSES,), lambda i: (0,)),
        ],
        out_specs=pl.BlockSpec((_MLP_BLK, N_CLASSES), lambda i: (i, 0)),
        out_shape=jax.ShapeDtypeStruct((N_PAD, N_CLASSES), jnp.float32),
    )(x_pad, W0, b0, W1, b1, W2, b2)


def _axpy_chunk(p_ref, a_ref, w_ref):
    def body(r, carry):
        for h in range(COLS // LANES):
            sl = pl.ds(h * LANES, LANES)
            w_ref[r, sl] = p_ref[r, sl] * w_ref[r, sl] + a_ref[r, sl]
        return carry

    lax.fori_loop(0, ZROWS, body, 0)


@functools.partial(
    pl.kernel,
    out_type=[jax.ShapeDtypeStruct((NC, N_PAD, COLS), jnp.float32),
              jax.ShapeDtypeStruct((NC, N_PAD, COLS), jnp.float32)],
    mesh=_mesh,
    compiler_params=_sc_params,
    scratch_types=[
        pltpu.VMEM_SHARED((N_PAD, COLS), jnp.float32),
        pltpu.VMEM_SHARED((N_PAD, COLS), jnp.float32),
        pltpu.VMEM((NB, EB), jnp.int32),
        pltpu.VMEM((NB, EB), jnp.int32),
        [pltpu.VMEM((EB, COLS), jnp.float32)] * 8,
        pltpu.VMEM((ZROWS, COLS), jnp.float32),
        pltpu.VMEM((ZROWS, COLS), jnp.float32),
        pltpu.VMEM((ZROWS, COLS), jnp.float32),
        pltpu.VMEM((ZROWS, COLS), jnp.float32),
        pltpu.SemaphoreType.DMA((8,)),
        pltpu.SemaphoreType.DMA((8,)),
    ],
)
def _prop_kernel(src_hbm, dst_hbm, g0_hbm, a0_hbm, p2_hbm, q2_hbm, h0a_hbm,
                 out_hbm, gw_hbm, g_sp, agg_sp, src_v, dst_v, gat, w_v, p_v,
                 a_v, z_v, gsem, ssem):
    c = lax.axis_index("c")
    s = lax.axis_index("s")
    r0 = s * ROWS
    rows_sl = pl.ds(r0, ROWS)
    nz = ROWS // ZROWS

    pltpu.sync_copy(src_hbm.at[s], src_v)
    pltpu.sync_copy(dst_hbm.at[s], dst_v)
    _zero_vmem(z_v, ZROWS, COLS)
    pltpu.sync_copy(g0_hbm.at[c, rows_sl], g_sp.at[rows_sl])

    def zero_body(z, carry):
        pltpu.sync_copy(z_v, agg_sp.at[pl.ds(r0 + z * ZROWS, ZROWS)])
        return carry

    lax.fori_loop(0, nz, zero_body, 0)
    plsc.subcore_barrier()

    N_HBM_SLOTS = 2

    def edge_phase(hbm_src):
        def g_src(b, j):
            if b < N_HBM_SLOTS:
                return hbm_src.at[c, :, :].at[src_v.at[j]]
            return g_sp.at[src_v.at[j]]

        def g_issue(j, b):
            pltpu.async_copy(g_src(b, j), gat[b], gsem.at[b])

        def g_wait(b):
            pltpu.make_async_copy(g_src(b, 0), gat[b], gsem.at[b]).wait()

        def s_issue(j, b):
            pltpu.async_copy(gat[b], agg_sp.at[dst_v.at[j]], ssem.at[b],
                             add=True)

        def s_wait(b):
            pltpu.make_async_copy(gat[b], agg_sp.at[dst_v.at[0]],
                                  ssem.at[b]).wait()

        for b in range(4):
            g_issue(b, b)
        for b in range(8):
            g_wait(b)
            s_issue(b, b)
            if b >= 4:
                s_wait(b - 4)
            g_issue(b + 4, (b + 4) % 8)

        def group(g, carry):
            for b in range(8):
                v = 8 * g + b
                g_wait(b)
                s_issue(v, b)
                s_wait((b + 4) % 8)
                g_issue(jnp.minimum(v + 4, NB - 1), (b + 4) % 8)
            return carry

        lax.fori_loop(1, NB // 8, group, 0)
        for b in range(4):
            g_wait(b)
        for b in range(4, 8):
            s_wait(b)

    for k in range(K):
        edge_phase(g0_hbm if k == 0 else gw_hbm)
        plsc.subcore_barrier()
        last = k == K - 1
        pm_hbm = q2_hbm if last else p2_hbm
        am_hbm = h0a_hbm if last else a0_hbm

        def upd_body(z, carry, last=last, pm_hbm=pm_hbm, am_hbm=am_hbm):
            zsl = pl.ds(r0 + z * ZROWS, ZROWS)
            pltpu.sync_copy(agg_sp.at[zsl], w_v)
            if not last:
                pltpu.sync_copy(z_v, agg_sp.at[zsl])
            pltpu.sync_copy(pm_hbm.at[c, zsl], p_v)
            pltpu.sync_copy(am_hbm.at[c, zsl], a_v)
            _axpy_chunk(p_v, a_v, w_v)
            if last:
                pltpu.sync_copy(w_v, out_hbm.at[c, zsl])
            else:
                pltpu.sync_copy(w_v, g_sp.at[zsl])
                pltpu.sync_copy(w_v, gw_hbm.at[c, zsl])
            return carry

        lax.fori_loop(0, nz, upd_body, 0)
        if not last:
            plsc.subcore_barrier()


def kernel(features, edge_index, W0, b0, W1, b1, W2, b2):
    f32 = jnp.float32
    src = edge_index[0]
    dst = edge_index[1]
    pad_e = jnp.full((E_PAD - N_EDGES,), N_NODES, jnp.int32)
    src_t = jnp.concatenate([src, pad_e]).reshape(NS, NB, EB)
    dst_t = jnp.concatenate([dst, pad_e]).reshape(NS, NB, EB)

    degs = _degrees_kernel(jnp.stack([src_t, dst_t]))
    out_deg = degs[0]
    in_deg = degs[1]
    out_norm = jnp.power(jnp.clip(out_deg, 1.0, None), -0.5)
    in_norm = jnp.power(jnp.clip(in_deg, 1.0, None), -0.5)

    x_pad = jnp.zeros((N_PAD, IN_FEATS), f32).at[:N_NODES].set(features)
    h0 = _mlp(x_pad, W0, b0, W1, b1, W2, b2)
    mask = (jnp.arange(N_PAD) < N_NODES).astype(f32)
    h0 = h0 * mask[:, None]

    g0 = h0 * out_norm[:, None]
    a0 = ALPHA * g0
    p2 = jnp.broadcast_to(((1.0 - ALPHA) * out_norm * in_norm)[:, None],
                          (N_PAD, N_CLASSES))
    q2 = jnp.broadcast_to(((1.0 - ALPHA) * in_norm)[:, None],
                          (N_PAD, N_CLASSES))
    h0a = ALPHA * h0

    def split_cols(x):
        return jnp.stack([x[:, :COLS], x[:, COLS:]])

    out, _ = _prop_kernel(src_t, dst_t, split_cols(g0), split_cols(a0),
                          split_cols(p2), split_cols(q2), split_cols(h0a))
    return jnp.concatenate([out[0, :N_NODES], out[1, :N_NODES]], axis=1)


# speedup vs baseline: 2.5143x; 1.4986x over previous
import functools

import jax
import jax.numpy as jnp
from jax import lax
from jax.experimental import pallas as pl
from jax.experimental.pallas import tpu as pltpu
from jax.experimental.pallas import tpu_sc as plsc

N_NODES = 10000
N_EDGES = 320000
IN_FEATS = 128
HIDDEN = 128
N_CLASSES = 64
K = 10
ALPHA = 0.1

NC = 2
NS = 16
LANES = 16
N_PAD = 10240
ROWS = N_PAD // NS
EB = 128
NB = 160
EPT = NB * EB
E_PAD = NS * EPT
COLS = N_CLASSES // NC
ZROWS = 64
NZ = ROWS // ZROWS

_mesh = plsc.VectorSubcoreMesh(core_axis_name="c", subcore_axis_name="s")
_sc_params = pltpu.CompilerParams(use_tc_tiling_on_sc=False)


def _zero_vmem(ref, rows, cols):
    for r in range(rows):
        for h in range(cols // LANES):
            ref[r, pl.ds(h * LANES, LANES)] = jnp.zeros((LANES,), jnp.float32)


@functools.partial(
    pl.kernel,
    out_type=jax.ShapeDtypeStruct((NC, N_PAD), jnp.float32),
    mesh=_mesh,
    compiler_params=_sc_params,
    scratch_types=[
        pltpu.VMEM_SHARED((N_PAD,), jnp.float32),
        pltpu.VMEM((NB, EB), jnp.int32),
        pltpu.VMEM((EB,), jnp.float32),
        pltpu.VMEM((ROWS,), jnp.float32),
    ],
)
def _degrees_kernel(idx_hbm, out_hbm, deg_sp, idx_v, ones_v, stage_v):
    c = lax.axis_index("c")
    s = lax.axis_index("s")
    r0 = s * ROWS
    for h in range(EB // LANES):
        ones_v[pl.ds(h * LANES, LANES)] = jnp.ones((LANES,), jnp.float32)
    for h in range(ROWS // LANES):
        stage_v[pl.ds(h * LANES, LANES)] = jnp.zeros((LANES,), jnp.float32)
    pltpu.sync_copy(stage_v, deg_sp.at[pl.ds(r0, ROWS)])
    pltpu.sync_copy(idx_hbm.at[c, s], idx_v)
    plsc.subcore_barrier()

    def body(j, carry):
        pltpu.sync_copy(ones_v, deg_sp.at[idx_v.at[j]], add=True)
        return carry

    lax.fori_loop(0, NB, body, 0)
    plsc.subcore_barrier()
    pltpu.sync_copy(deg_sp.at[pl.ds(r0, ROWS)], stage_v)
    pltpu.sync_copy(stage_v, out_hbm.at[c, pl.ds(r0, ROWS)])


def _mlp_body(x_ref, w0_ref, b0_ref, w1_ref, b1_ref, w2_ref, b2_ref, o_ref):
    x = x_ref[...]
    h = jnp.dot(x, w0_ref[...], preferred_element_type=jnp.float32) + b0_ref[...]
    h = jnp.maximum(h, 0.0)
    h = jnp.dot(h, w1_ref[...], preferred_element_type=jnp.float32) + b1_ref[...]
    h = jnp.maximum(h, 0.0)
    o_ref[...] = jnp.dot(h, w2_ref[...], preferred_element_type=jnp.float32) + b2_ref[...]


_MLP_BLK = 1024


def _mlp(x_pad, W0, b0, W1, b1, W2, b2):
    grid = (N_PAD // _MLP_BLK,)
    return pl.pallas_call(
        _mlp_body,
        grid=grid,
        in_specs=[
            pl.BlockSpec((_MLP_BLK, IN_FEATS), lambda i: (i, 0)),
            pl.BlockSpec((IN_FEATS, HIDDEN), lambda i: (0, 0)),
            pl.BlockSpec((HIDDEN,), lambda i: (0,)),
            pl.BlockSpec((HIDDEN, HIDDEN), lambda i: (0, 0)),
            pl.BlockSpec((HIDDEN,), lambda i: (0,)),
            pl.BlockSpec((HIDDEN, N_CLASSES), lambda i: (0, 0)),
            pl.BlockSpec((N_CLASSES,), lambda i: (0,)),
        ],
        out_specs=pl.BlockSpec((_MLP_BLK, N_CLASSES), lambda i: (i, 0)),
        out_shape=jax.ShapeDtypeStruct((N_PAD, N_CLASSES), jnp.float32),
    )(x_pad, W0, b0, W1, b1, W2, b2)


@functools.partial(
    pl.kernel,
    out_type=jax.ShapeDtypeStruct((NC, N_PAD, COLS), jnp.float32),
    mesh=_mesh,
    compiler_params=_sc_params,
    scratch_types=[
        pltpu.VMEM_SHARED((N_PAD, COLS), jnp.float32),
        pltpu.VMEM_SHARED((N_PAD, COLS), jnp.float32),
        pltpu.VMEM((NB, EB), jnp.int32),
        pltpu.VMEM((NB, EB), jnp.int32),
        [pltpu.VMEM((EB, COLS), jnp.float32)] * 8,
        [pltpu.VMEM((ZROWS, COLS), jnp.float32)] * 2,
        [pltpu.VMEM((ZROWS, COLS), jnp.float32)] * 2,
        [pltpu.VMEM((ZROWS, COLS), jnp.float32)] * 2,
        pltpu.VMEM((ZROWS, COLS), jnp.float32),
        pltpu.SemaphoreType.DMA((8,)),
        pltpu.SemaphoreType.DMA((8,)),
    ],
)
def _prop_kernel(src_hbm, dst_hbm, g0_hbm, a0_hbm, p2_hbm, q2_hbm, h0a_hbm,
                 out_hbm, g_sp, agg_sp, src_v, dst_v, gat, W, P, A, z_v,
                 gsem, ssem):
    c = lax.axis_index("c")
    s = lax.axis_index("s")
    r0 = s * ROWS
    rows_sl = pl.ds(r0, ROWS)

    pltpu.sync_copy(src_hbm.at[s], src_v)
    pltpu.sync_copy(dst_hbm.at[s], dst_v)
    _zero_vmem(z_v, ZROWS, COLS)
    pltpu.sync_copy(g0_hbm.at[c, rows_sl], g_sp.at[rows_sl])

    def zsl(z):
        return pl.ds(r0 + z * ZROWS, ZROWS)

    for z in range(NZ):
        pltpu.sync_copy(z_v, agg_sp.at[zsl(z)])
    plsc.subcore_barrier()

    def g_issue(j, b):
        pltpu.async_copy(g_sp.at[src_v.at[j]], gat[b], gsem.at[b])

    def g_wait(b):
        pltpu.make_async_copy(g_sp.at[src_v.at[0]], gat[b], gsem.at[b]).wait()

    def s_issue(j, b):
        pltpu.async_copy(gat[b], agg_sp.at[dst_v.at[j]], ssem.at[b], add=True)

    def s_wait(b):
        pltpu.make_async_copy(gat[b], agg_sp.at[dst_v.at[0]],
                              ssem.at[b]).wait()

    def edge_phase():
        for b in range(4):
            g_issue(b, b)
        for b in range(8):
            g_wait(b)
            s_issue(b, b)
            if b >= 4:
                s_wait(b - 4)
            g_issue(b + 4, (b + 4) % 8)

        def group(g, carry):
            for b in range(8):
                v = 8 * g + b
                g_wait(b)
                s_issue(v, b)
                s_wait((b + 4) % 8)
                g_issue(jnp.minimum(v + 4, NB - 1), (b + 4) % 8)
            return carry

        lax.fori_loop(1, NB // 8, group, 0)
        for b in range(4):
            g_wait(b)
        for b in range(4, 8):
            s_wait(b)

    def _axpy(p_ref, a_ref, w_ref):
        def body(r, carry):
            for h in range(COLS // LANES):
                sl = pl.ds(h * LANES, LANES)
                w_ref[r, sl] = p_ref[r, sl] * w_ref[r, sl] + a_ref[r, sl]
            return carry

        lax.fori_loop(0, ZROWS, body, 0)

    def blend_phase(last):
        pm = q2_hbm if last else p2_hbm
        am = h0a_hbm if last else a0_hbm
        dst = out_hbm if last else None

        def l_issue(z, par):
            pltpu.async_copy(agg_sp.at[zsl(z)], W[par], gsem.at[par])
            pltpu.async_copy(pm.at[c, zsl(z)], P[par], gsem.at[2 + par])
            pltpu.async_copy(am.at[c, zsl(z)], A[par], gsem.at[4 + par])

        def l_wait(par):
            pltpu.make_async_copy(agg_sp.at[zsl(0)], W[par],
                                  gsem.at[par]).wait()
            pltpu.make_async_copy(pm.at[c, zsl(0)], P[par],
                                  gsem.at[2 + par]).wait()
            pltpu.make_async_copy(am.at[c, zsl(0)], A[par],
                                  gsem.at[4 + par]).wait()

        def st_issue(z, par):
            if last:
                pltpu.async_copy(W[par], dst.at[c, zsl(z)], ssem.at[par])
            else:
                pltpu.async_copy(W[par], g_sp.at[zsl(z)], ssem.at[par])

        def st_wait(par):
            if last:
                pltpu.make_async_copy(W[par], dst.at[c, zsl(0)],
                                      ssem.at[par]).wait()
            else:
                pltpu.make_async_copy(W[par], g_sp.at[zsl(0)],
                                      ssem.at[par]).wait()

        def zero_issue(z, par):
            pltpu.async_copy(z_v, agg_sp.at[zsl(z)], ssem.at[2 + par])

        def zero_wait(par):
            pltpu.make_async_copy(z_v, agg_sp.at[zsl(0)],
                                  ssem.at[2 + par]).wait()

        l_issue(0, 0)
        for z in range(NZ):
            par = z % 2
            if z + 1 < NZ:
                if z >= 1:
                    st_wait(1 - par)
                    if not last:
                        zero_wait(1 - par)
                l_issue(z + 1, 1 - par)
            l_wait(par)
            if not last:
                zero_issue(z, par)
            _axpy(P[par], A[par], W[par])
            st_issue(z, par)
        for par in range(2):
            st_wait(par)
            if not last:
                zero_wait(par)

    def step_body(kk, carry):
        edge_phase()
        plsc.subcore_barrier()
        blend_phase(last=False)
        plsc.subcore_barrier()
        return carry

    lax.fori_loop(0, K - 1, step_body, 0)
    edge_phase()
    plsc.subcore_barrier()
    blend_phase(last=True)


def kernel(features, edge_index, W0, b0, W1, b1, W2, b2):
    f32 = jnp.float32
    src = edge_index[0]
    dst = edge_index[1]
    pad_e = jnp.full((E_PAD - N_EDGES,), N_NODES, jnp.int32)
    src_t = jnp.concatenate([src, pad_e]).reshape(NS, NB, EB)
    dst_t = jnp.concatenate([dst, pad_e]).reshape(NS, NB, EB)

    degs = _degrees_kernel(jnp.stack([src_t, dst_t]))
    out_deg = degs[0]
    in_deg = degs[1]
    out_norm = jnp.power(jnp.clip(out_deg, 1.0, None), -0.5)
    in_norm = jnp.power(jnp.clip(in_deg, 1.0, None), -0.5)

    x_pad = jnp.zeros((N_PAD, IN_FEATS), f32).at[:N_NODES].set(features)
    h0 = _mlp(x_pad, W0, b0, W1, b1, W2, b2)
    mask = (jnp.arange(N_PAD) < N_NODES).astype(f32)
    h0 = h0 * mask[:, None]

    g0 = h0 * out_norm[:, None]
    a0 = ALPHA * g0
    p2 = jnp.broadcast_to(((1.0 - ALPHA) * out_norm * in_norm)[:, None],
                          (N_PAD, N_CLASSES))
    q2 = jnp.broadcast_to(((1.0 - ALPHA) * in_norm)[:, None],
                          (N_PAD, N_CLASSES))
    h0a = ALPHA * h0

    def split_cols(x):
        return jnp.stack([x[:, :COLS], x[:, COLS:]])

    out = _prop_kernel(src_t, dst_t, split_cols(g0), split_cols(a0),
                       split_cols(p2), split_cols(q2), split_cols(h0a))
    return jnp.concatenate([out[0, :N_NODES], out[1, :N_NODES]], axis=1)


# speedup vs baseline: 2.7558x; 1.0960x over previous
import functools

import jax
import jax.numpy as jnp
from jax import lax
from jax.experimental import pallas as pl
from jax.experimental.pallas import tpu as pltpu
from jax.experimental.pallas import tpu_sc as plsc

N_NODES = 10000
N_EDGES = 320000
IN_FEATS = 128
HIDDEN = 128
N_CLASSES = 64
K = 10
ALPHA = 0.1

NC = 2
NS = 16
LANES = 16
N_PAD = 10240
ROWS = N_PAD // NS
EB = 128
NB = 160
EPT = NB * EB
E_PAD = NS * EPT
COLS = N_CLASSES // NC
ZROWS = 64
NZ = ROWS // ZROWS

_mesh = plsc.VectorSubcoreMesh(core_axis_name="c", subcore_axis_name="s")
_sc_params = pltpu.CompilerParams(use_tc_tiling_on_sc=False)


def _zero_vmem(ref, rows, cols):
    for r in range(rows):
        for h in range(cols // LANES):
            ref[r, pl.ds(h * LANES, LANES)] = jnp.zeros((LANES,), jnp.float32)


@functools.partial(
    pl.kernel,
    out_type=jax.ShapeDtypeStruct((NC, N_PAD), jnp.float32),
    mesh=_mesh,
    compiler_params=_sc_params,
    scratch_types=[
        pltpu.VMEM_SHARED((N_PAD,), jnp.float32),
        pltpu.VMEM((NB, EB), jnp.int32),
        pltpu.VMEM((EB,), jnp.float32),
        pltpu.VMEM((ROWS,), jnp.float32),
    ],
)
def _degrees_kernel(idx_hbm, out_hbm, deg_sp, idx_v, ones_v, stage_v):
    c = lax.axis_index("c")
    s = lax.axis_index("s")
    r0 = s * ROWS
    for h in range(EB // LANES):
        ones_v[pl.ds(h * LANES, LANES)] = jnp.ones((LANES,), jnp.float32)
    for h in range(ROWS // LANES):
        stage_v[pl.ds(h * LANES, LANES)] = jnp.zeros((LANES,), jnp.float32)
    pltpu.sync_copy(stage_v, deg_sp.at[pl.ds(r0, ROWS)])
    pltpu.sync_copy(idx_hbm.at[c, s], idx_v)
    plsc.subcore_barrier()

    def body(j, carry):
        pltpu.sync_copy(ones_v, deg_sp.at[idx_v.at[j]], add=True)
        return carry

    lax.fori_loop(0, NB, body, 0)
    plsc.subcore_barrier()
    pltpu.sync_copy(deg_sp.at[pl.ds(r0, ROWS)], stage_v)
    pltpu.sync_copy(stage_v, out_hbm.at[c, pl.ds(r0, ROWS)])


def _mlp_body(x_ref, w0_ref, b0_ref, w1_ref, b1_ref, w2_ref, b2_ref,
              od_ref, id_ref, h0_ref, g0_ref, p_ref, q_ref):
    x = x_ref[...]
    h = jnp.dot(x, w0_ref[...], preferred_element_type=jnp.float32) + b0_ref[...]
    h = jnp.maximum(h, 0.0)
    h = jnp.dot(h, w1_ref[...], preferred_element_type=jnp.float32) + b1_ref[...]
    h = jnp.maximum(h, 0.0)
    h = jnp.dot(h, w2_ref[...], preferred_element_type=jnp.float32) + b2_ref[...]
    i = pl.program_id(0)
    row = i * _MLP_BLK + jax.lax.broadcasted_iota(jnp.int32, (_MLP_BLK, 1), 0)
    h = jnp.where(row < N_NODES, h, 0.0)
    onorm = jax.lax.rsqrt(jnp.clip(od_ref[...], 1.0, None))
    inorm = jax.lax.rsqrt(jnp.clip(id_ref[...], 1.0, None))
    h0_ref[...] = h
    g0_ref[...] = h * onorm[:, None]
    p_ref[...] = jnp.broadcast_to(((1.0 - ALPHA) * onorm * inorm)[:, None],
                                  (_MLP_BLK, N_CLASSES))
    q_ref[...] = jnp.broadcast_to(((1.0 - ALPHA) * inorm)[:, None],
                                  (_MLP_BLK, N_CLASSES))


_MLP_BLK = 1024


def _mlp(x_pad, W0, b0, W1, b1, W2, b2, out_deg, in_deg):
    grid = (N_PAD // _MLP_BLK,)
    full = lambda i: (0, 0)
    return pl.pallas_call(
        _mlp_body,
        grid=grid,
        in_specs=[
            pl.BlockSpec((_MLP_BLK, IN_FEATS), lambda i: (i, 0)),
            pl.BlockSpec((IN_FEATS, HIDDEN), full),
            pl.BlockSpec((HIDDEN,), lambda i: (0,)),
            pl.BlockSpec((HIDDEN, HIDDEN), full),
            pl.BlockSpec((HIDDEN,), lambda i: (0,)),
            pl.BlockSpec((HIDDEN, N_CLASSES), full),
            pl.BlockSpec((N_CLASSES,), lambda i: (0,)),
            pl.BlockSpec((_MLP_BLK,), lambda i: (i,)),
            pl.BlockSpec((_MLP_BLK,), lambda i: (i,)),
        ],
        out_specs=[
            pl.BlockSpec((_MLP_BLK, N_CLASSES), lambda i: (i, 0)),
            pl.BlockSpec((_MLP_BLK, N_CLASSES), lambda i: (i, 0)),
            pl.BlockSpec((_MLP_BLK, N_CLASSES), lambda i: (i, 0)),
            pl.BlockSpec((_MLP_BLK, N_CLASSES), lambda i: (i, 0)),
        ],
        out_shape=[
            jax.ShapeDtypeStruct((N_PAD, N_CLASSES), jnp.float32),
            jax.ShapeDtypeStruct((N_PAD, N_CLASSES), jnp.float32),
            jax.ShapeDtypeStruct((N_PAD, N_CLASSES), jnp.float32),
            jax.ShapeDtypeStruct((N_PAD, N_CLASSES), jnp.float32),
        ],
    )(x_pad, W0, b0, W1, b1, W2, b2, out_deg, in_deg)


@functools.partial(
    pl.kernel,
    out_type=jax.ShapeDtypeStruct((N_PAD, N_CLASSES), jnp.float32),
    mesh=_mesh,
    compiler_params=_sc_params,
    scratch_types=[
        pltpu.VMEM_SHARED((N_PAD, COLS), jnp.float32),
        pltpu.VMEM_SHARED((N_PAD, COLS), jnp.float32),
        pltpu.VMEM((NB, EB), jnp.int32),
        pltpu.VMEM((NB, EB), jnp.int32),
        [pltpu.VMEM((EB, COLS), jnp.float32)] * 8,
        [pltpu.VMEM((ZROWS, COLS), jnp.float32)] * 2,
        [pltpu.VMEM((ZROWS, COLS), jnp.float32)] * 2,
        [pltpu.VMEM((ZROWS, COLS), jnp.float32)] * 2,
        pltpu.VMEM((ZROWS, COLS), jnp.float32),
        pltpu.SemaphoreType.DMA((8,)),
        pltpu.SemaphoreType.DMA((8,)),
    ],
)
def _prop_kernel(src_hbm, dst_hbm, g0_hbm, h0_hbm, p_hbm, q_hbm,
                 out_hbm, g_sp, agg_sp, src_v, dst_v, gat, W, P, A, z_v,
                 gsem, ssem):
    c = lax.axis_index("c")
    s = lax.axis_index("s")
    r0 = s * ROWS
    rows_sl = pl.ds(r0, ROWS)
    csl = pl.ds(c * COLS, COLS)

    pltpu.sync_copy(src_hbm.at[s], src_v)
    pltpu.sync_copy(dst_hbm.at[s], dst_v)
    _zero_vmem(z_v, ZROWS, COLS)
    pltpu.sync_copy(g0_hbm.at[rows_sl, csl], g_sp.at[rows_sl])

    def zsl(z):
        return pl.ds(r0 + z * ZROWS, ZROWS)

    for z in range(NZ):
        pltpu.sync_copy(z_v, agg_sp.at[zsl(z)])
    plsc.subcore_barrier()

    def g_issue(j, b):
        pltpu.async_copy(g_sp.at[src_v.at[j]], gat[b], gsem.at[b])

    def g_wait(b):
        pltpu.make_async_copy(g_sp.at[src_v.at[0]], gat[b], gsem.at[b]).wait()

    def s_issue(j, b):
        pltpu.async_copy(gat[b], agg_sp.at[dst_v.at[j]], ssem.at[b], add=True)

    def s_wait(b):
        pltpu.make_async_copy(gat[b], agg_sp.at[dst_v.at[0]],
                              ssem.at[b]).wait()

    def edge_phase():
        for b in range(4):
            g_issue(b, b)
        for b in range(8):
            g_wait(b)
            s_issue(b, b)
            if b >= 4:
                s_wait(b - 4)
            g_issue(b + 4, (b + 4) % 8)

        def group(g, carry):
            for b in range(8):
                v = 8 * g + b
                g_wait(b)
                s_issue(v, b)
                s_wait((b + 4) % 8)
                g_issue(jnp.minimum(v + 4, NB - 1), (b + 4) % 8)
            return carry

        lax.fori_loop(1, NB // 8, group, 0)
        for b in range(4):
            g_wait(b)
        for b in range(4, 8):
            s_wait(b)

    def _axpy(p_ref, a_ref, w_ref):
        def body(r, carry):
            for h in range(COLS // LANES):
                sl = pl.ds(h * LANES, LANES)
                w_ref[r, sl] = p_ref[r, sl] * w_ref[r, sl] + ALPHA * a_ref[r, sl]
            return carry

        lax.fori_loop(0, ZROWS, body, 0)

    def blend_phase(last):
        pm = q_hbm if last else p_hbm
        am = h0_hbm if last else g0_hbm

        def l_issue(z, par):
            pltpu.async_copy(agg_sp.at[zsl(z)], W[par], gsem.at[par])
            pltpu.async_copy(pm.at[zsl(z), csl], P[par], gsem.at[2 + par])
            pltpu.async_copy(am.at[zsl(z), csl], A[par], gsem.at[4 + par])

        def l_wait(par):
            pltpu.make_async_copy(agg_sp.at[zsl(0)], W[par],
                                  gsem.at[par]).wait()
            pltpu.make_async_copy(pm.at[zsl(0), csl], P[par],
                                  gsem.at[2 + par]).wait()
            pltpu.make_async_copy(am.at[zsl(0), csl], A[par],
                                  gsem.at[4 + par]).wait()

        def st_issue(z, par):
            if last:
                pltpu.async_copy(W[par], out_hbm.at[zsl(z), csl],
                                 ssem.at[par])
            else:
                pltpu.async_copy(W[par], g_sp.at[zsl(z)], ssem.at[par])

        def st_wait(par):
            if last:
                pltpu.make_async_copy(W[par], out_hbm.at[zsl(0), csl],
                                      ssem.at[par]).wait()
            else:
                pltpu.make_async_copy(W[par], g_sp.at[zsl(0)],
                                      ssem.at[par]).wait()

        def zero_issue(z, par):
            pltpu.async_copy(z_v, agg_sp.at[zsl(z)], ssem.at[2 + par])

        def zero_wait(par):
            pltpu.make_async_copy(z_v, agg_sp.at[zsl(0)],
                                  ssem.at[2 + par]).wait()

        l_issue(0, 0)
        for z in range(NZ):
            par = z % 2
            if z + 1 < NZ:
                if z >= 1:
                    st_wait(1 - par)
                    if not last:
                        zero_wait(1 - par)
                l_issue(z + 1, 1 - par)
            l_wait(par)
            if not last:
                zero_issue(z, par)
            _axpy(P[par], A[par], W[par])
            st_issue(z, par)
        for par in range(2):
            st_wait(par)
            if not last:
                zero_wait(par)

    def step_body(kk, carry):
        edge_phase()
        plsc.subcore_barrier()
        blend_phase(last=False)
        plsc.subcore_barrier()
        return carry

    lax.fori_loop(0, K - 1, step_body, 0)
    edge_phase()
    plsc.subcore_barrier()
    blend_phase(last=True)


def kernel(features, edge_index, W0, b0, W1, b1, W2, b2):
    f32 = jnp.float32
    src = edge_index[0]
    dst = edge_index[1]
    pad_e = jnp.full((E_PAD - N_EDGES,), N_NODES, jnp.int32)
    src_t = jnp.concatenate([src, pad_e]).reshape(NS, NB, EB)
    dst_t = jnp.concatenate([dst, pad_e]).reshape(NS, NB, EB)

    degs = _degrees_kernel(jnp.stack([src_t, dst_t]))

    x_pad = jnp.zeros((N_PAD, IN_FEATS), f32).at[:N_NODES].set(features)
    h0, g0, pvec, qvec = _mlp(x_pad, W0, b0, W1, b1, W2, b2,
                              degs[0], degs[1])

    out = _prop_kernel(src_t, dst_t, g0, h0, pvec, qvec)
    return out[:N_NODES]


# speedup vs baseline: 2.7824x; 1.0097x over previous
import functools

import jax
import jax.numpy as jnp
from jax import lax
from jax.experimental import pallas as pl
from jax.experimental.pallas import tpu as pltpu
from jax.experimental.pallas import tpu_sc as plsc

N_NODES = 10000
N_EDGES = 320000
IN_FEATS = 128
HIDDEN = 128
N_CLASSES = 64
K = 10
ALPHA = 0.1

NC = 2
NS = 16
LANES = 16
N_PAD = 10240
ROWS = N_PAD // NS
EB = 128
NB = 160
EPT = NB * EB
E_PAD = NS * EPT
COLS = N_CLASSES // NC
ZROWS = 64
NZ = ROWS // ZROWS

_mesh = plsc.VectorSubcoreMesh(core_axis_name="c", subcore_axis_name="s")
_sc_params = pltpu.CompilerParams(use_tc_tiling_on_sc=False)


def _zero_vmem(ref, rows, cols):
    for r in range(rows):
        for h in range(cols // LANES):
            ref[r, pl.ds(h * LANES, LANES)] = jnp.zeros((LANES,), jnp.float32)


@functools.partial(
    pl.kernel,
    out_type=jax.ShapeDtypeStruct((NC, N_PAD), jnp.float32),
    mesh=_mesh,
    compiler_params=_sc_params,
    scratch_types=[
        pltpu.VMEM_SHARED((N_PAD,), jnp.float32),
        pltpu.VMEM((NB, EB), jnp.int32),
        pltpu.VMEM((EB,), jnp.float32),
        pltpu.VMEM((ROWS,), jnp.float32),
        pltpu.SemaphoreType.DMA((8,)),
    ],
)
def _degrees_kernel(src_hbm, dst_hbm, out_hbm, deg_sp, idx_v, ones_v,
                    stage_v, sems):
    c = lax.axis_index("c")
    s = lax.axis_index("s")
    r0 = s * ROWS
    for h in range(EB // LANES):
        ones_v[pl.ds(h * LANES, LANES)] = jnp.ones((LANES,), jnp.float32)
    for h in range(ROWS // LANES):
        stage_v[pl.ds(h * LANES, LANES)] = jnp.zeros((LANES,), jnp.float32)
    pltpu.sync_copy(stage_v, deg_sp.at[pl.ds(r0, ROWS)])

    @pl.when(c == 0)
    def _():
        pltpu.sync_copy(src_hbm.at[s], idx_v)

    @pl.when(c == 1)
    def _():
        pltpu.sync_copy(dst_hbm.at[s], idx_v)

    plsc.subcore_barrier()

    def s_issue(j, b):
        pltpu.async_copy(ones_v, deg_sp.at[idx_v.at[j]], sems.at[b],
                         add=True)

    def s_wait(b):
        pltpu.make_async_copy(ones_v, deg_sp.at[idx_v.at[0]],
                              sems.at[b]).wait()

    for b in range(8):
        s_issue(b, b)

    def group(g, carry):
        for b in range(8):
            s_wait(b)
            s_issue(8 * g + b, b)
        return carry

    lax.fori_loop(1, NB // 8, group, 0)
    for b in range(8):
        s_wait(b)
    plsc.subcore_barrier()
    pltpu.sync_copy(deg_sp.at[pl.ds(r0, ROWS)], stage_v)
    pltpu.sync_copy(stage_v, out_hbm.at[c, pl.ds(r0, ROWS)])


def _mlp_body(x_ref, w0_ref, b0_ref, w1_ref, b1_ref, w2_ref, b2_ref,
              od_ref, id_ref, h0_ref, g0_ref, p_ref, q_ref):
    x = x_ref[...]
    h = jnp.dot(x, w0_ref[...], preferred_element_type=jnp.float32) + b0_ref[...]
    h = jnp.maximum(h, 0.0)
    h = jnp.dot(h, w1_ref[...], preferred_element_type=jnp.float32) + b1_ref[...]
    h = jnp.maximum(h, 0.0)
    h = jnp.dot(h, w2_ref[...], preferred_element_type=jnp.float32) + b2_ref[...]
    i = pl.program_id(0)
    row = i * _MLP_BLK + jax.lax.broadcasted_iota(jnp.int32, (_MLP_BLK, 1), 0)
    h = jnp.where(row < N_NODES, h, 0.0)
    onorm = jax.lax.rsqrt(jnp.clip(od_ref[...], 1.0, None))
    inorm = jax.lax.rsqrt(jnp.clip(id_ref[...], 1.0, None))
    h0_ref[...] = h
    g0_ref[...] = h * onorm[:, None]
    p_ref[...] = jnp.broadcast_to(((1.0 - ALPHA) * onorm * inorm)[:, None],
                                  (_MLP_BLK, N_CLASSES))
    q_ref[...] = jnp.broadcast_to(((1.0 - ALPHA) * inorm)[:, None],
                                  (_MLP_BLK, N_CLASSES))


_MLP_BLK = 1024


def _mlp(x_pad, W0, b0, W1, b1, W2, b2, out_deg, in_deg):
    grid = (N_PAD // _MLP_BLK,)
    full = lambda i: (0, 0)
    return pl.pallas_call(
        _mlp_body,
        grid=grid,
        in_specs=[
            pl.BlockSpec((_MLP_BLK, IN_FEATS), lambda i: (i, 0)),
            pl.BlockSpec((IN_FEATS, HIDDEN), full),
            pl.BlockSpec((HIDDEN,), lambda i: (0,)),
            pl.BlockSpec((HIDDEN, HIDDEN), full),
            pl.BlockSpec((HIDDEN,), lambda i: (0,)),
            pl.BlockSpec((HIDDEN, N_CLASSES), full),
            pl.BlockSpec((N_CLASSES,), lambda i: (0,)),
            pl.BlockSpec((_MLP_BLK,), lambda i: (i,)),
            pl.BlockSpec((_MLP_BLK,), lambda i: (i,)),
        ],
        out_specs=[
            pl.BlockSpec((_MLP_BLK, N_CLASSES), lambda i: (i, 0)),
            pl.BlockSpec((_MLP_BLK, N_CLASSES), lambda i: (i, 0)),
            pl.BlockSpec((_MLP_BLK, N_CLASSES), lambda i: (i, 0)),
            pl.BlockSpec((_MLP_BLK, N_CLASSES), lambda i: (i, 0)),
        ],
        out_shape=[
            jax.ShapeDtypeStruct((N_PAD, N_CLASSES), jnp.float32),
            jax.ShapeDtypeStruct((N_PAD, N_CLASSES), jnp.float32),
            jax.ShapeDtypeStruct((N_PAD, N_CLASSES), jnp.float32),
            jax.ShapeDtypeStruct((N_PAD, N_CLASSES), jnp.float32),
        ],
    )(x_pad, W0, b0, W1, b1, W2, b2, out_deg, in_deg)


@functools.partial(
    pl.kernel,
    out_type=jax.ShapeDtypeStruct((N_PAD, N_CLASSES), jnp.float32),
    mesh=_mesh,
    compiler_params=_sc_params,
    scratch_types=[
        pltpu.VMEM_SHARED((N_PAD, COLS), jnp.float32),
        pltpu.VMEM_SHARED((N_PAD, COLS), jnp.float32),
        pltpu.VMEM((NB, EB), jnp.int32),
        pltpu.VMEM((NB, EB), jnp.int32),
        [pltpu.VMEM((EB, COLS), jnp.float32)] * 8,
        [pltpu.VMEM((ZROWS, COLS), jnp.float32)] * 2,
        [pltpu.VMEM((ZROWS, COLS), jnp.float32)] * 2,
        [pltpu.VMEM((ZROWS, COLS), jnp.float32)] * 2,
        pltpu.VMEM((ZROWS, COLS), jnp.float32),
        pltpu.SemaphoreType.DMA((8,)),
        pltpu.SemaphoreType.DMA((8,)),
    ],
)
def _prop_kernel(src_hbm, dst_hbm, g0_hbm, h0_hbm, p_hbm, q_hbm,
                 out_hbm, g_sp, agg_sp, src_v, dst_v, gat, W, P, A, z_v,
                 gsem, ssem):
    c = lax.axis_index("c")
    s = lax.axis_index("s")
    r0 = s * ROWS
    rows_sl = pl.ds(r0, ROWS)
    csl = pl.ds(c * COLS, COLS)

    pltpu.sync_copy(src_hbm.at[s], src_v)
    pltpu.sync_copy(dst_hbm.at[s], dst_v)
    _zero_vmem(z_v, ZROWS, COLS)
    pltpu.sync_copy(g0_hbm.at[rows_sl, csl], g_sp.at[rows_sl])

    def zsl(z):
        return pl.ds(r0 + z * ZROWS, ZROWS)

    for z in range(NZ):
        pltpu.sync_copy(z_v, agg_sp.at[zsl(z)])
    plsc.subcore_barrier()

    def g_issue(j, b):
        pltpu.async_copy(g_sp.at[src_v.at[j]], gat[b], gsem.at[b])

    def g_wait(b):
        pltpu.make_async_copy(g_sp.at[src_v.at[0]], gat[b], gsem.at[b]).wait()

    def s_issue(j, b):
        pltpu.async_copy(gat[b], agg_sp.at[dst_v.at[j]], ssem.at[b], add=True)

    def s_wait(b):
        pltpu.make_async_copy(gat[b], agg_sp.at[dst_v.at[0]],
                              ssem.at[b]).wait()

    def edge_phase():
        for b in range(4):
            g_issue(b, b)
        for b in range(8):
            g_wait(b)
            s_issue(b, b)
            if b >= 4:
                s_wait(b - 4)
            g_issue(b + 4, (b + 4) % 8)

        def group(g, carry):
            for b in range(8):
                v = 8 * g + b
                g_wait(b)
                s_issue(v, b)
                s_wait((b + 4) % 8)
                g_issue(jnp.minimum(v + 4, NB - 1), (b + 4) % 8)
            return carry

        lax.fori_loop(1, NB // 8, group, 0)
        for b in range(4):
            g_wait(b)
        for b in range(4, 8):
            s_wait(b)

    def _axpy(p_ref, a_ref, w_ref):
        def body(r, carry):
            for h in range(COLS // LANES):
                sl = pl.ds(h * LANES, LANES)
                w_ref[r, sl] = p_ref[r, sl] * w_ref[r, sl] + ALPHA * a_ref[r, sl]
            return carry

        lax.fori_loop(0, ZROWS, body, 0)

    def blend_phase(last):
        pm = q_hbm if last else p_hbm
        am = h0_hbm if last else g0_hbm

        def l_issue(z, par):
            pltpu.async_copy(agg_sp.at[zsl(z)], W[par], gsem.at[par])
            pltpu.async_copy(pm.at[zsl(z), csl], P[par], gsem.at[2 + par])
            pltpu.async_copy(am.at[zsl(z), csl], A[par], gsem.at[4 + par])

        def l_wait(par):
            pltpu.make_async_copy(agg_sp.at[zsl(0)], W[par],
                                  gsem.at[par]).wait()
            pltpu.make_async_copy(pm.at[zsl(0), csl], P[par],
                                  gsem.at[2 + par]).wait()
            pltpu.make_async_copy(am.at[zsl(0), csl], A[par],
                                  gsem.at[4 + par]).wait()

        def st_issue(z, par):
            if last:
                pltpu.async_copy(W[par], out_hbm.at[zsl(z), csl],
                                 ssem.at[par])
            else:
                pltpu.async_copy(W[par], g_sp.at[zsl(z)], ssem.at[par])

        def st_wait(par):
            if last:
                pltpu.make_async_copy(W[par], out_hbm.at[zsl(0), csl],
                                      ssem.at[par]).wait()
            else:
                pltpu.make_async_copy(W[par], g_sp.at[zsl(0)],
                                      ssem.at[par]).wait()

        def zero_issue(z, par):
            pltpu.async_copy(z_v, agg_sp.at[zsl(z)], ssem.at[2 + par])

        def zero_wait(par):
            pltpu.make_async_copy(z_v, agg_sp.at[zsl(0)],
                                  ssem.at[2 + par]).wait()

        l_issue(0, 0)
        for z in range(NZ):
            par = z % 2
            if z + 1 < NZ:
                if z >= 1:
                    st_wait(1 - par)
                    if not last:
                        zero_wait(1 - par)
                l_issue(z + 1, 1 - par)
            l_wait(par)
            if not last:
                zero_issue(z, par)
            _axpy(P[par], A[par], W[par])
            st_issue(z, par)
        for par in range(2):
            st_wait(par)
            if not last:
                zero_wait(par)

    def step_body(kk, carry):
        edge_phase()
        plsc.subcore_barrier()
        blend_phase(last=False)
        plsc.subcore_barrier()
        return carry

    lax.fori_loop(0, K - 1, step_body, 0)
    edge_phase()
    plsc.subcore_barrier()
    blend_phase(last=True)


def kernel(features, edge_index, W0, b0, W1, b1, W2, b2):
    f32 = jnp.float32
    src = edge_index[0]
    dst = edge_index[1]
    pad_e = jnp.full((E_PAD - N_EDGES,), N_NODES, jnp.int32)
    src_t = jnp.concatenate([src, pad_e]).reshape(NS, NB, EB)
    dst_t = jnp.concatenate([dst, pad_e]).reshape(NS, NB, EB)

    degs = _degrees_kernel(src_t, dst_t)

    h0, g0, pvec, qvec = _mlp(features, W0, b0, W1, b1, W2, b2,
                              degs[0], degs[1])

    out = _prop_kernel(src_t, dst_t, g0, h0, pvec, qvec)
    return out[:N_NODES]
